# baseline (speedup 1.0000x reference)
import functools

import jax
import jax.numpy as jnp
from jax import lax
from jax.experimental import pallas as pl
from jax.experimental.pallas import tpu as pltpu
from jax.experimental.pallas import tpu_sc as plsc

_B = 16384
_D = 32
_H1, _H2 = 64, 32
_NU = 1000000
_NM = 100000

_C = 64
_IDX_BATCH = 128
_WAVE = 16


def _sc_gather_all(user, movie, actor_flat, country_flat, type_flat,
                   ut_1d, mt_1d, actor_tab, country_tab, type_tab):
  info = plsc.get_sparse_core_info()
  nw = info.num_cores * info.num_subcores
  bw = _B // nw
  nchunk = bw // _C

  mesh = plsc.VectorSubcoreMesh(core_axis_name="c", subcore_axis_name="s")
  out_row = jax.ShapeDtypeStruct((_B, _D), jnp.float32)
  out_colT = jax.ShapeDtypeStruct((_D, _B), jnp.float32)

  nbatch = bw * _D // _IDX_BATCH
  nwave = nbatch // _WAVE

  @functools.partial(
      pl.kernel,
      mesh=mesh,
      out_type=[out_colT, out_colT, out_row, out_row, out_row],
      compiler_params=pltpu.CompilerParams(use_tc_tiling_on_sc=False),
      scratch_types=[
          pltpu.VMEM((bw,), jnp.int32),
          pltpu.VMEM((bw,), jnp.int32),
          pltpu.VMEM((_WAVE * _IDX_BATCH,), jnp.int32),
          pltpu.VMEM((bw * _D,), jnp.float32),
          pltpu.VMEM((bw * _D,), jnp.float32),
          pltpu.VMEM((_C * 20,), jnp.int32),
          pltpu.VMEM((_C * 4,), jnp.int32),
          pltpu.VMEM((_C * 4,), jnp.int32),
          pltpu.VMEM((_C * 20, _D), jnp.float32),
          pltpu.VMEM((_C * 4, _D), jnp.float32),
          pltpu.VMEM((_C * 4, _D), jnp.float32),
          pltpu.VMEM((_C, _D), jnp.float32),
          pltpu.VMEM((_C, _D), jnp.float32),
          pltpu.VMEM((_C, _D), jnp.float32),
          pltpu.SemaphoreType.DMA,
      ],
  )
  def body(user_i, movie_i, actor_i, country_i, type_i,
           ut, mt, at_, ct, tt,
           uo, mo, ao, co, to,
           idx_u, idx_m, stage, ubuf, mbuf,
           idx_a, idx_c, idx_t,
           rows_a, rows_c, rows_t,
           pool_a, pool_c, pool_t, sem):
    wid = lax.axis_index("s") * info.num_cores + lax.axis_index("c")
    base = wid * bw

    def elem_gather(tab, n_rows, idx_ref, buf):
      def wave(w, _):
        cps = []
        for j in range(_WAVE):
          k = w * _WAVE + j
          d = k >> 2
          soff = (k & 3) * _IDX_BATCH
          st = j * _IDX_BATCH
          for v in range(_IDX_BATCH // 16):
            stage[pl.ds(st + v * 16, 16)] = (
                idx_ref[pl.ds(soff + v * 16, 16)] + d * n_rows)
          cps.append(pltpu.async_copy(
              tab.at[stage.at[pl.ds(st, _IDX_BATCH)]],
              buf.at[pl.ds(k * _IDX_BATCH, _IDX_BATCH)], sem))
        for cp in cps:
          cp.wait()
        return 0
      lax.fori_loop(0, nwave, wave, 0)

    pltpu.sync_copy(user_i.at[pl.ds(base, bw)], idx_u)
    pltpu.sync_copy(movie_i.at[pl.ds(base, bw)], idx_m)
    elem_gather(ut, _NU, idx_u, ubuf)
    elem_gather(mt, _NM, idx_m, mbuf)

    def col_out(buf, out):
      def row(d, _):
        pltpu.sync_copy(buf.at[pl.ds(d * bw, bw)],
                        out.at[d, pl.ds(base, bw)])
        return 0
      lax.fori_loop(0, _D, row, 0)

    col_out(ubuf, uo)
    col_out(mbuf, mo)

    def gather(tab, idx_ref, rows_ref, n):
      copies = []
      for off in range(0, n, _IDX_BATCH):
        sz = min(_IDX_BATCH, n - off)
        copies.append(pltpu.async_copy(
            tab.at[idx_ref.at[pl.ds(off, sz)]],
            rows_ref.at[pl.ds(off, sz)], sem))
      return copies

    def pool_loop(rows, pool, n, scale):
      def samp(c, _):
        r = c * n
        a0 = rows[r, pl.ds(0, 16)]
        a1 = rows[r, pl.ds(16, 16)]
        for j in range(1, n):
          a0 = a0 + rows[r + j, pl.ds(0, 16)]
          a1 = a1 + rows[r + j, pl.ds(16, 16)]
        pool[c, pl.ds(0, 16)] = a0 * scale
        pool[c, pl.ds(16, 16)] = a1 * scale
        return 0
      lax.fori_loop(0, _C, samp, 0)

    def chunk(k, _):
      cb = base + k * _C
      pltpu.sync_copy(actor_i.at[pl.ds(cb * 20, _C * 20)], idx_a)
      pltpu.sync_copy(country_i.at[pl.ds(cb * 4, _C * 4)], idx_c)
      pltpu.sync_copy(type_i.at[pl.ds(cb * 4, _C * 4)], idx_t)

      ca = gather(at_, idx_a, rows_a, _C * 20)
      cc = gather(ct, idx_c, rows_c, _C * 4)
      ctp = gather(tt, idx_t, rows_t, _C * 4)

      for c in ca:
        c.wait()
      pool_loop(rows_a, pool_a, 20, 1.0 / 20.0)
      for c in cc:
        c.wait()
      pool_loop(rows_c, pool_c, 4, 0.25)
      for c in ctp:
        c.wait()
      pool_loop(rows_t, pool_t, 4, 0.25)

      pltpu.sync_copy(pool_a, ao.at[pl.ds(cb, _C)])
      pltpu.sync_copy(pool_c, co.at[pl.ds(cb, _C)])
      pltpu.sync_copy(pool_t, to.at[pl.ds(cb, _C)])
      return 0

    lax.fori_loop(0, nchunk, chunk, 0)

  return body(user, movie, actor_flat, country_flat, type_flat,
              ut_1d, mt_1d, actor_tab, country_tab, type_tab)


_TB = 512

_DN0 = (((0,), (0,)), ((), ()))
_DN1 = (((0,), (1,)), ((), ()))


def _mlp_body(ut, mt, a, c, t, s, w1u, w1m, w1a, w1c, w1t, w1s, b1,
              w2, b2, w3, b3, o):
  h = (lax.dot_general(w1u[...], ut[...], _DN0)
       + lax.dot_general(w1m[...], mt[...], _DN0)
       + lax.dot_general(w1a[...], a[...], _DN1)
       + lax.dot_general(w1c[...], c[...], _DN1)
       + lax.dot_general(w1t[...], t[...], _DN1)
       + lax.dot_general(w1s[...], s[...], _DN1)
       + b1[...])
  h = jnp.maximum(h, 0.0)
  h = jnp.maximum(lax.dot_general(w2[...], h, _DN0) + b2[...], 0.0)
  o[...] = lax.dot_general(w3[...], h, _DN0) + b3[...]


def _mlp(ut, mt, a, c, t, s, W1, b1, W2, b2, W3, b3):
  grid = _B // _TB
  featT_spec = pl.BlockSpec((_D, _TB), lambda i: (0, i))
  feat_spec = pl.BlockSpec((_TB, _D), lambda i: (i, 0))
  s_spec = pl.BlockSpec((_TB, 4), lambda i: (i, 0))

  def full(shape):
    return pl.BlockSpec(shape, lambda i: tuple(0 for _ in shape))

  w1u, w1m, w1a, w1c, w1t = (W1[k * _D:(k + 1) * _D] for k in range(5))
  w1s = W1[5 * _D:]
  out = pl.pallas_call(
      _mlp_body,
      grid=(grid,),
      in_specs=[featT_spec] * 2 + [feat_spec] * 3 + [s_spec] + [
          full((_D, _H1))] * 5 + [full((4, _H1)), full((_H1, 1)),
          full((_H1, _H2)), full((_H2, 1)), full((_H2, 1)), full((1, 1))],
      out_specs=pl.BlockSpec((1, _TB), lambda i: (0, i)),
      out_shape=jax.ShapeDtypeStruct((1, _B), jnp.float32),
  )(ut, mt, a, c, t, s, w1u, w1m, w1a, w1c, w1t, w1s, b1.reshape(_H1, 1),
    W2, b2.reshape(_H2, 1), W3, b3.reshape(1, 1))
  return out


@jax.jit
def kernel(user, movie, actor, country, movie_type, num_reviews,
           normalized_rating, useful_ratings, useless_ratings,
           user_table, movie_table, actor_table, country_table,
           movie_type_table, W1, b1, W2, b2, W3, b3):
  user = user.astype(jnp.int32)
  movie = movie.astype(jnp.int32)
  actor_flat = actor.reshape(-1).astype(jnp.int32)
  country_flat = country.reshape(-1).astype(jnp.int32)
  type_flat = movie_type.reshape(-1).astype(jnp.int32)

  ut_1d = user_table.T.reshape(-1)
  mt_1d = movie_table.T.reshape(-1)

  ut, mt, a, c, t = _sc_gather_all(
      user, movie, actor_flat, country_flat, type_flat,
      ut_1d, mt_1d, actor_table, country_table, movie_type_table)

  s = jnp.stack([num_reviews, normalized_rating, useful_ratings,
                 useless_ratings], axis=1)
  out = _mlp(ut, mt, a, c, t, s, W1, b1, W2, b2, W3, b3)
  return out.reshape(-1)


# speedup vs baseline: 3.9613x; 3.9613x over previous
import functools

import jax
import jax.numpy as jnp
from jax import lax
from jax.experimental import pallas as pl
from jax.experimental.pallas import tpu as pltpu
from jax.experimental.pallas import tpu_sc as plsc

_B = 16384
_D = 32
_H1, _H2 = 64, 32

_C = 64
_IDX_BATCH = 128


def _sc_gather_pool(user, movie, actor_flat, country_flat, type_flat,
                    user_tab, movie_tab, actor_tab, country_tab, type_tab):
  info = plsc.get_sparse_core_info()
  nw = info.num_cores * info.num_subcores
  bw = _B // nw
  nchunk = bw // _C

  mesh = plsc.VectorSubcoreMesh(core_axis_name="c", subcore_axis_name="s")

  out_t = jax.ShapeDtypeStruct((_B, _D), jnp.float32)

  @functools.partial(
      pl.kernel,
      mesh=mesh,
      out_type=[out_t] * 5,
      compiler_params=pltpu.CompilerParams(use_tc_tiling_on_sc=False),
      scratch_types=[
          pltpu.VMEM((_C,), jnp.int32),
          pltpu.VMEM((_C,), jnp.int32),
          pltpu.VMEM((_C * 20,), jnp.int32),
          pltpu.VMEM((_C * 4,), jnp.int32),
          pltpu.VMEM((_C * 4,), jnp.int32),
          pltpu.VMEM((_C, _D), jnp.float32),
          pltpu.VMEM((_C, _D), jnp.float32),
          pltpu.VMEM((_C * 20, _D), jnp.float32),
          pltpu.VMEM((_C * 4, _D), jnp.float32),
          pltpu.VMEM((_C * 4, _D), jnp.float32),
          pltpu.VMEM((_C, _D), jnp.float32),
          pltpu.VMEM((_C, _D), jnp.float32),
          pltpu.VMEM((_C, _D), jnp.float32),
          pltpu.SemaphoreType.DMA,
      ],
  )
  def body(user_i, movie_i, actor_i, country_i, type_i,
           ut, mt, at_, ct, tt,
           uo, mo, ao, co, to,
           idx_u, idx_m, idx_a, idx_c, idx_t,
           rows_u, rows_m, rows_a, rows_c, rows_t,
           pool_a, pool_c, pool_t, sem):
    wid = lax.axis_index("s") * info.num_cores + lax.axis_index("c")
    base = wid * bw

    def gather(tab, idx_ref, rows_ref, n):
      copies = []
      for off in range(0, n, _IDX_BATCH):
        sz = min(_IDX_BATCH, n - off)
        copies.append(pltpu.async_copy(
            tab.at[idx_ref.at[pl.ds(off, sz)]],
            rows_ref.at[pl.ds(off, sz)], sem))
      return copies

    def pool_loop(rows, pool, n, scale):
      def samp(c, _):
        r = c * n
        a0 = rows[r, pl.ds(0, 16)]
        a1 = rows[r, pl.ds(16, 16)]
        for j in range(1, n):
          a0 = a0 + rows[r + j, pl.ds(0, 16)]
          a1 = a1 + rows[r + j, pl.ds(16, 16)]
        pool[c, pl.ds(0, 16)] = a0 * scale
        pool[c, pl.ds(16, 16)] = a1 * scale
        return 0
      lax.fori_loop(0, _C, samp, 0)

    def chunk(k, _):
      cb = base + k * _C
      pltpu.sync_copy(user_i.at[pl.ds(cb, _C)], idx_u)
      pltpu.sync_copy(movie_i.at[pl.ds(cb, _C)], idx_m)
      pltpu.sync_copy(actor_i.at[pl.ds(cb * 20, _C * 20)], idx_a)
      pltpu.sync_copy(country_i.at[pl.ds(cb * 4, _C * 4)], idx_c)
      pltpu.sync_copy(type_i.at[pl.ds(cb * 4, _C * 4)], idx_t)

      cu = gather(ut, idx_u, rows_u, _C)
      cm = gather(mt, idx_m, rows_m, _C)
      ca = gather(at_, idx_a, rows_a, _C * 20)
      cc = gather(ct, idx_c, rows_c, _C * 4)
      ctp = gather(tt, idx_t, rows_t, _C * 4)

      for c in cu:
        c.wait()
      pltpu.sync_copy(rows_u, uo.at[pl.ds(cb, _C)])
      for c in cm:
        c.wait()
      pltpu.sync_copy(rows_m, mo.at[pl.ds(cb, _C)])
      for c in ca:
        c.wait()
      pool_loop(rows_a, pool_a, 20, 1.0 / 20.0)
      for c in cc:
        c.wait()
      pool_loop(rows_c, pool_c, 4, 0.25)
      for c in ctp:
        c.wait()
      pool_loop(rows_t, pool_t, 4, 0.25)

      pltpu.sync_copy(pool_a, ao.at[pl.ds(cb, _C)])
      pltpu.sync_copy(pool_c, co.at[pl.ds(cb, _C)])
      pltpu.sync_copy(pool_t, to.at[pl.ds(cb, _C)])
      return 0

    lax.fori_loop(0, nchunk, chunk, 0)

  return body(user, movie, actor_flat, country_flat, type_flat,
              user_tab, movie_tab, actor_tab, country_tab, type_tab)


_TB = 512


def _mlp_body(u, m, a, c, t, s, w1u, w1m, w1a, w1c, w1t, w1s, b1,
              w2, b2, w3, b3, o):
  h = (jnp.dot(u[...], w1u[...])
       + jnp.dot(m[...], w1m[...])
       + jnp.dot(a[...], w1a[...])
       + jnp.dot(c[...], w1c[...])
       + jnp.dot(t[...], w1t[...])
       + jnp.dot(s[...], w1s[...])
       + b1[...])
  h = jnp.maximum(h, 0.0)
  h = jnp.maximum(jnp.dot(h, w2[...]) + b2[...], 0.0)
  o[...] = jnp.dot(h, w3[...]) + b3[...]


def _mlp(u, m, a, c, t, s, W1, b1, W2, b2, W3, b3):
  grid = _B // _TB
  feat_spec = pl.BlockSpec((_TB, _D), lambda i: (i, 0))
  s_spec = pl.BlockSpec((_TB, 4), lambda i: (i, 0))

  def full(shape):
    return pl.BlockSpec(shape, lambda i: tuple(0 for _ in shape))

  w1u, w1m, w1a, w1c, w1t = (W1[k * _D:(k + 1) * _D] for k in range(5))
  w1s = W1[5 * _D:]
  out = pl.pallas_call(
      _mlp_body,
      grid=(grid,),
      in_specs=[feat_spec] * 5 + [s_spec] + [
          full((_D, _H1))] * 5 + [full((4, _H1)), full((1, _H1)),
          full((_H1, _H2)), full((1, _H2)), full((_H2, 1)), full((1, 1))],
      out_specs=pl.BlockSpec((_TB, 1), lambda i: (i, 0)),
      out_shape=jax.ShapeDtypeStruct((_B, 1), jnp.float32),
  )(u, m, a, c, t, s, w1u, w1m, w1a, w1c, w1t, w1s, b1.reshape(1, _H1),
    W2, b2.reshape(1, _H2), W3, b3.reshape(1, 1))
  return out


@jax.jit
def kernel(user, movie, actor, country, movie_type, num_reviews,
           normalized_rating, useful_ratings, useless_ratings,
           user_table, movie_table, actor_table, country_table,
           movie_type_table, W1, b1, W2, b2, W3, b3):
  user = user.astype(jnp.int32)
  movie = movie.astype(jnp.int32)
  actor_flat = actor.reshape(-1).astype(jnp.int32)
  country_flat = country.reshape(-1).astype(jnp.int32)
  type_flat = movie_type.reshape(-1).astype(jnp.int32)

  u, m, a, c, t = _sc_gather_pool(
      user, movie, actor_flat, country_flat, type_flat,
      user_table, movie_table, actor_table, country_table,
      movie_type_table)

  s = jnp.stack([num_reviews, normalized_rating, useful_ratings,
                 useless_ratings], axis=1)
  out = _mlp(u, m, a, c, t, s, W1, b1, W2, b2, W3, b3)
  return jnp.squeeze(out, axis=-1)


# speedup vs baseline: 7.0024x; 1.7677x over previous
import functools

import jax
import jax.numpy as jnp
from jax import lax
from jax.experimental import pallas as pl
from jax.experimental.pallas import tpu as pltpu
from jax.experimental.pallas import tpu_sc as plsc

_B = 16384
_D = 32
_H1, _H2 = 64, 32
_NU = 1000000

_C = 64
_IDX_BATCH = 128

_TPC = _NU // 128 + 1
_NFULL = 4 * (_TPC - 1)
_UROWS = 4 * _TPC * 8
_WSLOTS = 16


def _sc_relayout_user(ut_t, u_rag):
  info = plsc.get_sparse_core_info()
  nw = info.num_cores * info.num_subcores
  nwave = (_NFULL + nw * _WSLOTS - 1) // (nw * _WSLOTS)

  mesh = plsc.VectorSubcoreMesh(core_axis_name="c", subcore_axis_name="s")

  @functools.partial(
      pl.kernel,
      mesh=mesh,
      out_type=jax.ShapeDtypeStruct((_UROWS, 128), jnp.float32),
      compiler_params=pltpu.CompilerParams(use_tc_tiling_on_sc=True),
      scratch_types=[
          pltpu.VMEM((8 * _WSLOTS, 128), jnp.float32),
          pltpu.VMEM((_D, 128), jnp.float32),
          pltpu.SemaphoreType.DMA,
      ],
  )
  def body(ut, ur, uo, buf, tbuf, sem):
    wid = lax.axis_index("s") * info.num_cores + lax.axis_index("c")

    def wave(q, _):
      tiles = []
      cps = []
      for j in range(_WSLOTS):
        i = (q * _WSLOTS + j) * nw + wid
        i = jnp.minimum(i, _NFULL - 1)
        k = i // (_TPC - 1)
        tc = i % (_TPC - 1)
        tiles.append((k, tc))
        cps.append(pltpu.async_copy(
            ut.at[pl.ds(pl.multiple_of(k * 8, 8), 8),
                  pl.ds(pl.multiple_of(tc * 128, 128), 128)],
            buf.at[pl.ds(j * 8, 8)], sem))
      for cp in cps:
        cp.wait()
      cps = []
      for j in range(_WSLOTS):
        k, tc = tiles[j]
        row = (k * _TPC + tc) * 8
        cps.append(pltpu.async_copy(
            buf.at[pl.ds(j * 8, 8)],
            uo.at[pl.ds(pl.multiple_of(row, 8), 8)], sem))
      for cp in cps:
        cp.wait()
      return 0

    lax.fori_loop(0, nwave, wave, 0)

    @pl.when(wid == 0)
    def _():
      pltpu.sync_copy(ur, tbuf)
      for k in range(4):
        pltpu.sync_copy(tbuf.at[pl.ds(k * 8, 8)],
                        uo.at[pl.ds((k * _TPC + _TPC - 1) * 8, 8)])

  return body(ut_t, u_rag)


def _sc_gather_all(user, movie, actor_flat, country_flat, type_flat,
                   ut_1d, movie_tab, actor_tab, country_tab, type_tab):
  info = plsc.get_sparse_core_info()
  nw = info.num_cores * info.num_subcores
  bw = _B // nw
  nchunk = bw // _C

  mesh = plsc.VectorSubcoreMesh(core_axis_name="c", subcore_axis_name="s")
  out_row = jax.ShapeDtypeStruct((_B, _D), jnp.float32)
  out_colT = jax.ShapeDtypeStruct((_D, _B), jnp.float32)

  nbatch = bw * _D // _IDX_BATCH
  nwave = nbatch // 16

  @functools.partial(
      pl.kernel,
      mesh=mesh,
      out_type=[out_colT, out_row, out_row, out_row, out_row],
      compiler_params=pltpu.CompilerParams(use_tc_tiling_on_sc=False),
      scratch_types=[
          pltpu.VMEM((bw,), jnp.int32),
          pltpu.VMEM((16 * _IDX_BATCH,), jnp.int32),
          pltpu.VMEM((bw * _D,), jnp.float32),
          pltpu.VMEM((_C,), jnp.int32),
          pltpu.VMEM((_C * 20,), jnp.int32),
          pltpu.VMEM((_C * 4,), jnp.int32),
          pltpu.VMEM((_C * 4,), jnp.int32),
          pltpu.VMEM((_C, _D), jnp.float32),
          pltpu.VMEM((_C * 20, _D), jnp.float32),
          pltpu.VMEM((_C * 4, _D), jnp.float32),
          pltpu.VMEM((_C * 4, _D), jnp.float32),
          pltpu.VMEM((_C, _D), jnp.float32),
          pltpu.VMEM((_C, _D), jnp.float32),
          pltpu.VMEM((_C, _D), jnp.float32),
          pltpu.SemaphoreType.DMA,
      ],
  )
  def body(user_i, movie_i, actor_i, country_i, type_i,
           ut, mt, at_, ct, tt,
           uo, mo, ao, co, to,
           idx_u, stage, ubuf,
           idx_m, idx_a, idx_c, idx_t,
           rows_m, rows_a, rows_c, rows_t,
           pool_a, pool_c, pool_t, sem):
    wid = lax.axis_index("s") * info.num_cores + lax.axis_index("c")
    base = wid * bw

    pltpu.sync_copy(user_i.at[pl.ds(base, bw)], idx_u)

    def tform(v, _):
      x = idx_u[pl.ds(v * 16, 16)]
      idx_u[pl.ds(v * 16, 16)] = (
          lax.shift_left(lax.shift_right_logical(x, 7), 10)
          + (x & 127))
      return 0
    lax.fori_loop(0, bw // 16, tform, 0)

    def uwave(w, _):
      cps = []
      for j in range(16):
        k = w * 16 + j
        d = k >> 2
        cst = (d >> 3) * (_TPC * 1024) + (d & 7) * 128
        soff = (k & 3) * _IDX_BATCH
        st = j * _IDX_BATCH
        for v in range(_IDX_BATCH // 16):
          stage[pl.ds(st + v * 16, 16)] = idx_u[pl.ds(soff + v * 16, 16)] + cst
        cps.append(pltpu.async_copy(
            ut.at[stage.at[pl.ds(st, _IDX_BATCH)]],
            ubuf.at[pl.ds(k * _IDX_BATCH, _IDX_BATCH)], sem))
      for cp in cps:
        cp.wait()
      return 0
    lax.fori_loop(0, nwave, uwave, 0)

    def urow(d, _):
      pltpu.sync_copy(ubuf.at[pl.ds(d * bw, bw)], uo.at[d, pl.ds(base, bw)])
      return 0
    lax.fori_loop(0, _D, urow, 0)

    def gather(tab, idx_ref, rows_ref, n):
      copies = []
      for off in range(0, n, _IDX_BATCH):
        sz = min(_IDX_BATCH, n - off)
        copies.append(pltpu.async_copy(
            tab.at[idx_ref.at[pl.ds(off, sz)]],
            rows_ref.at[pl.ds(off, sz)], sem))
      return copies

    def pool_loop(rows, pool, n, scale):
      def samp(c, _):
        r = c * n
        a0 = rows[r, pl.ds(0, 16)]
        a1 = rows[r, pl.ds(16, 16)]
        for j in range(1, n):
          a0 = a0 + rows[r + j, pl.ds(0, 16)]
          a1 = a1 + rows[r + j, pl.ds(16, 16)]
        pool[c, pl.ds(0, 16)] = a0 * scale
        pool[c, pl.ds(16, 16)] = a1 * scale
        return 0
      lax.fori_loop(0, _C, samp, 0)

    def chunk(k, _):
      cb = base + k * _C
      pltpu.sync_copy(movie_i.at[pl.ds(cb, _C)], idx_m)
      pltpu.sync_copy(actor_i.at[pl.ds(cb * 20, _C * 20)], idx_a)
      pltpu.sync_copy(country_i.at[pl.ds(cb * 4, _C * 4)], idx_c)
      pltpu.sync_copy(type_i.at[pl.ds(cb * 4, _C * 4)], idx_t)

      cm = gather(mt, idx_m, rows_m, _C)
      ca = gather(at_, idx_a, rows_a, _C * 20)
      cc = gather(ct, idx_c, rows_c, _C * 4)
      ctp = gather(tt, idx_t, rows_t, _C * 4)

      for c in cm:
        c.wait()
      pltpu.sync_copy(rows_m, mo.at[pl.ds(cb, _C)])
      for c in ca:
        c.wait()
      pool_loop(rows_a, pool_a, 20, 1.0 / 20.0)
      for c in cc:
        c.wait()
      pool_loop(rows_c, pool_c, 4, 0.25)
      for c in ctp:
        c.wait()
      pool_loop(rows_t, pool_t, 4, 0.25)

      pltpu.sync_copy(pool_a, ao.at[pl.ds(cb, _C)])
      pltpu.sync_copy(pool_c, co.at[pl.ds(cb, _C)])
      pltpu.sync_copy(pool_t, to.at[pl.ds(cb, _C)])
      return 0

    lax.fori_loop(0, nchunk, chunk, 0)

  return body(user, movie, actor_flat, country_flat, type_flat,
              ut_1d, movie_tab, actor_tab, country_tab, type_tab)


_TB = 512

_DN0 = (((0,), (0,)), ((), ()))
_DN1 = (((0,), (1,)), ((), ()))


def _mlp_body(ut, m, a, c, t, s, w1u, w1m, w1a, w1c, w1t, w1s, b1,
              w2, b2, w3, b3, o):
  h = (lax.dot_general(w1u[...], ut[...], _DN0)
       + lax.dot_general(w1m[...], m[...], _DN1)
       + lax.dot_general(w1a[...], a[...], _DN1)
       + lax.dot_general(w1c[...], c[...], _DN1)
       + lax.dot_general(w1t[...], t[...], _DN1)
       + lax.dot_general(w1s[...], s[...], _DN1)
       + b1[...])
  h = jnp.maximum(h, 0.0)
  h = jnp.maximum(lax.dot_general(w2[...], h, _DN0) + b2[...], 0.0)
  o[...] = lax.dot_general(w3[...], h, _DN0) + b3[...]


def _mlp(ut, m, a, c, t, s, W1, b1, W2, b2, W3, b3):
  grid = _B // _TB
  featT_spec = pl.BlockSpec((_D, _TB), lambda i: (0, i))
  feat_spec = pl.BlockSpec((_TB, _D), lambda i: (i, 0))
  s_spec = pl.BlockSpec((_TB, 4), lambda i: (i, 0))

  def full(shape):
    return pl.BlockSpec(shape, lambda i: tuple(0 for _ in shape))

  w1u, w1m, w1a, w1c, w1t = (W1[k * _D:(k + 1) * _D] for k in range(5))
  w1s = W1[5 * _D:]
  out = pl.pallas_call(
      _mlp_body,
      grid=(grid,),
      in_specs=[featT_spec] + [feat_spec] * 4 + [s_spec] + [
          full((_D, _H1))] * 5 + [full((4, _H1)), full((_H1, 1)),
          full((_H1, _H2)), full((_H2, 1)), full((_H2, 1)), full((1, 1))],
      out_specs=pl.BlockSpec((1, _TB), lambda i: (0, i)),
      out_shape=jax.ShapeDtypeStruct((1, _B), jnp.float32),
  )(ut, m, a, c, t, s, w1u, w1m, w1a, w1c, w1t, w1s, b1.reshape(_H1, 1),
    W2, b2.reshape(_H2, 1), W3, b3.reshape(1, 1))
  return out


@jax.jit
def kernel(user, movie, actor, country, movie_type, num_reviews,
           normalized_rating, useful_ratings, useless_ratings,
           user_table, movie_table, actor_table, country_table,
           movie_type_table, W1, b1, W2, b2, W3, b3):
  user = user.astype(jnp.int32)
  movie = movie.astype(jnp.int32)
  actor_flat = actor.reshape(-1).astype(jnp.int32)
  country_flat = country.reshape(-1).astype(jnp.int32)
  type_flat = movie_type.reshape(-1).astype(jnp.int32)

  u_rag = jnp.pad(user_table.T[:, _NU - _NU % 128:], ((0, 0), (0, 64)))
  u_lin = _sc_relayout_user(user_table.T, u_rag)
  ut_1d = u_lin.reshape(-1)

  ut, m, a, c, t = _sc_gather_all(
      user, movie, actor_flat, country_flat, type_flat,
      ut_1d, movie_table, actor_table, country_table, movie_type_table)

  s = jnp.stack([num_reviews, normalized_rating, useful_ratings,
                 useless_ratings], axis=1)
  out = _mlp(ut, m, a, c, t, s, W1, b1, W2, b2, W3, b3)
  return out.reshape(-1)


# speedup vs baseline: 7.7892x; 1.1124x over previous
import functools

import jax
import jax.numpy as jnp
from jax import lax
from jax.experimental import pallas as pl
from jax.experimental.pallas import tpu as pltpu
from jax.experimental.pallas import tpu_sc as plsc

_B = 16384
_D = 32
_H1, _H2 = 64, 32
_NU = 1000000

_C = 64
_IDX_BATCH = 128

_TPC = _NU // 128 + 1
_NFULL = 4 * (_TPC - 1)
_UROWS = 4 * _TPC * 8
_GT = 64
_GPC = (_TPC - 1) // _GT
_NG = 4 * _GPC
_NLEFT = _NFULL - _NG * _GT


def _sc_relayout_user(ut_t, u_rag):
  info = plsc.get_sparse_core_info()
  nw = info.num_cores * info.num_subcores
  ngrp = (_NG + nw - 1) // nw

  mesh = plsc.VectorSubcoreMesh(core_axis_name="c", subcore_axis_name="s")

  @functools.partial(
      pl.kernel,
      mesh=mesh,
      out_type=jax.ShapeDtypeStruct((_UROWS, 128), jnp.float32),
      compiler_params=pltpu.CompilerParams(use_tc_tiling_on_sc=True),
      scratch_types=[
          pltpu.VMEM((8, 128 * _GT), jnp.float32),
          pltpu.VMEM((_D, 128), jnp.float32),
          pltpu.SemaphoreType.DMA,
      ],
  )
  def body(ut, ur, uo, buf, tbuf, sem):
    wid = lax.axis_index("s") * info.num_cores + lax.axis_index("c")

    def grp(q, _):
      g = q * nw + wid
      g = jnp.minimum(g, _NG - 1)
      k = g // _GPC
      tc0 = (g % _GPC) * _GT
      pltpu.sync_copy(
          ut.at[pl.ds(pl.multiple_of(k * 8, 8), 8),
                pl.ds(pl.multiple_of(tc0 * 128, 128), 128 * _GT)], buf)
      row0 = (k * _TPC + tc0) * 8
      cps = []
      for t in range(_GT):
        cps.append(pltpu.async_copy(
            buf.at[:, pl.ds(t * 128, 128)],
            uo.at[pl.ds(pl.multiple_of(row0 + t * 8, 8), 8)], sem))
      for cp in cps:
        cp.wait()
      return 0

    lax.fori_loop(0, ngrp, grp, 0)

    nl_pc = _NLEFT // 4
    for w in range(_NLEFT):
      @pl.when(wid == w)
      def _(w=w):
        k = w // nl_pc
        tc = _GPC * _GT + w % nl_pc
        pltpu.sync_copy(
            ut.at[pl.ds(k * 8, 8), pl.ds(tc * 128, 128)],
            buf.at[:, pl.ds(0, 128)])
        pltpu.sync_copy(buf.at[:, pl.ds(0, 128)],
                        uo.at[pl.ds((k * _TPC + tc) * 8, 8)])

    @pl.when(wid == _NLEFT)
    def _():
      pltpu.sync_copy(ur, tbuf)
      for k in range(4):
        pltpu.sync_copy(tbuf.at[pl.ds(k * 8, 8)],
                        uo.at[pl.ds((k * _TPC + _TPC - 1) * 8, 8)])

  return body(ut_t, u_rag)


def _sc_gather_all(user, movie, actor_flat, country_flat, type_flat,
                   ut_1d, movie_tab, actor_tab, country_tab, type_tab):
  info = plsc.get_sparse_core_info()
  nw = info.num_cores * info.num_subcores
  bw = _B // nw
  nchunk = bw // _C

  mesh = plsc.VectorSubcoreMesh(core_axis_name="c", subcore_axis_name="s")
  out_row = jax.ShapeDtypeStruct((_B, _D), jnp.float32)
  out_colT = jax.ShapeDtypeStruct((_D, _B), jnp.float32)

  nbatch = bw * _D // _IDX_BATCH
  nwave = nbatch // 16

  @functools.partial(
      pl.kernel,
      mesh=mesh,
      out_type=[out_colT, out_row, out_row, out_row, out_row],
      compiler_params=pltpu.CompilerParams(use_tc_tiling_on_sc=False),
      scratch_types=[
          pltpu.VMEM((bw,), jnp.int32),
          pltpu.VMEM((16 * _IDX_BATCH,), jnp.int32),
          pltpu.VMEM((bw * _D,), jnp.float32),
          pltpu.VMEM((_C,), jnp.int32),
          pltpu.VMEM((_C * 20,), jnp.int32),
          pltpu.VMEM((_C * 4,), jnp.int32),
          pltpu.VMEM((_C * 4,), jnp.int32),
          pltpu.VMEM((_C, _D), jnp.float32),
          pltpu.VMEM((_C * 20, _D), jnp.float32),
          pltpu.VMEM((_C * 4, _D), jnp.float32),
          pltpu.VMEM((_C * 4, _D), jnp.float32),
          pltpu.VMEM((_C, _D), jnp.float32),
          pltpu.VMEM((_C, _D), jnp.float32),
          pltpu.VMEM((_C, _D), jnp.float32),
          pltpu.SemaphoreType.DMA,
      ],
  )
  def body(user_i, movie_i, actor_i, country_i, type_i,
           ut, mt, at_, ct, tt,
           uo, mo, ao, co, to,
           idx_u, stage, ubuf,
           idx_m, idx_a, idx_c, idx_t,
           rows_m, rows_a, rows_c, rows_t,
           pool_a, pool_c, pool_t, sem):
    wid = lax.axis_index("s") * info.num_cores + lax.axis_index("c")
    base = wid * bw

    pltpu.sync_copy(user_i.at[pl.ds(base, bw)], idx_u)

    def tform(v, _):
      x = idx_u[pl.ds(v * 16, 16)]
      idx_u[pl.ds(v * 16, 16)] = (
          lax.shift_left(lax.shift_right_logical(x, 7), 10)
          + (x & 127))
      return 0
    lax.fori_loop(0, bw // 16, tform, 0)

    def uwave(w, _):
      cps = []
      for j in range(16):
        k = w * 16 + j
        d = k >> 2
        cst = (d >> 3) * (_TPC * 1024) + (d & 7) * 128
        soff = (k & 3) * _IDX_BATCH
        st = j * _IDX_BATCH
        for v in range(_IDX_BATCH // 16):
          stage[pl.ds(st + v * 16, 16)] = idx_u[pl.ds(soff + v * 16, 16)] + cst
        cps.append(pltpu.async_copy(
            ut.at[stage.at[pl.ds(st, _IDX_BATCH)]],
            ubuf.at[pl.ds(k * _IDX_BATCH, _IDX_BATCH)], sem))
      for cp in cps:
        cp.wait()
      return 0
    lax.fori_loop(0, nwave, uwave, 0)

    def urow(d, _):
      pltpu.sync_copy(ubuf.at[pl.ds(d * bw, bw)], uo.at[d, pl.ds(base, bw)])
      return 0
    lax.fori_loop(0, _D, urow, 0)

    def gather(tab, idx_ref, rows_ref, n):
      copies = []
      for off in range(0, n, _IDX_BATCH):
        sz = min(_IDX_BATCH, n - off)
        copies.append(pltpu.async_copy(
            tab.at[idx_ref.at[pl.ds(off, sz)]],
            rows_ref.at[pl.ds(off, sz)], sem))
      return copies

    def pool_loop(rows, pool, n, scale):
      def samp(c, _):
        r = c * n
        a0 = rows[r, pl.ds(0, 16)]
        a1 = rows[r, pl.ds(16, 16)]
        for j in range(1, n):
          a0 = a0 + rows[r + j, pl.ds(0, 16)]
          a1 = a1 + rows[r + j, pl.ds(16, 16)]
        pool[c, pl.ds(0, 16)] = a0 * scale
        pool[c, pl.ds(16, 16)] = a1 * scale
        return 0
      lax.fori_loop(0, _C, samp, 0)

    def chunk(k, _):
      cb = base + k * _C
      pltpu.sync_copy(movie_i.at[pl.ds(cb, _C)], idx_m)
      pltpu.sync_copy(actor_i.at[pl.ds(cb * 20, _C * 20)], idx_a)
      pltpu.sync_copy(country_i.at[pl.ds(cb * 4, _C * 4)], idx_c)
      pltpu.sync_copy(type_i.at[pl.ds(cb * 4, _C * 4)], idx_t)

      cm = gather(mt, idx_m, rows_m, _C)
      ca = gather(at_, idx_a, rows_a, _C * 20)
      cc = gather(ct, idx_c, rows_c, _C * 4)
      ctp = gather(tt, idx_t, rows_t, _C * 4)

      for c in cm:
        c.wait()
      pltpu.sync_copy(rows_m, mo.at[pl.ds(cb, _C)])
      for c in ca:
        c.wait()
      pool_loop(rows_a, pool_a, 20, 1.0 / 20.0)
      for c in cc:
        c.wait()
      pool_loop(rows_c, pool_c, 4, 0.25)
      for c in ctp:
        c.wait()
      pool_loop(rows_t, pool_t, 4, 0.25)

      pltpu.sync_copy(pool_a, ao.at[pl.ds(cb, _C)])
      pltpu.sync_copy(pool_c, co.at[pl.ds(cb, _C)])
      pltpu.sync_copy(pool_t, to.at[pl.ds(cb, _C)])
      return 0

    lax.fori_loop(0, nchunk, chunk, 0)

  return body(user, movie, actor_flat, country_flat, type_flat,
              ut_1d, movie_tab, actor_tab, country_tab, type_tab)


_TB = 512

_DN0 = (((0,), (0,)), ((), ()))
_DN1 = (((0,), (1,)), ((), ()))


def _mlp_body(ut, m, a, c, t, s, w1u, w1m, w1a, w1c, w1t, w1s, b1,
              w2, b2, w3, b3, o):
  h = (lax.dot_general(w1u[...], ut[...], _DN0)
       + lax.dot_general(w1m[...], m[...], _DN1)
       + lax.dot_general(w1a[...], a[...], _DN1)
       + lax.dot_general(w1c[...], c[...], _DN1)
       + lax.dot_general(w1t[...], t[...], _DN1)
       + lax.dot_general(w1s[...], s[...], _DN1)
       + b1[...])
  h = jnp.maximum(h, 0.0)
  h = jnp.maximum(lax.dot_general(w2[...], h, _DN0) + b2[...], 0.0)
  o[...] = lax.dot_general(w3[...], h, _DN0) + b3[...]


def _mlp(ut, m, a, c, t, s, W1, b1, W2, b2, W3, b3):
  grid = _B // _TB
  featT_spec = pl.BlockSpec((_D, _TB), lambda i: (0, i))
  feat_spec = pl.BlockSpec((_TB, _D), lambda i: (i, 0))
  s_spec = pl.BlockSpec((_TB, 4), lambda i: (i, 0))

  def full(shape):
    return pl.BlockSpec(shape, lambda i: tuple(0 for _ in shape))

  w1u, w1m, w1a, w1c, w1t = (W1[k * _D:(k + 1) * _D] for k in range(5))
  w1s = W1[5 * _D:]
  out = pl.pallas_call(
      _mlp_body,
      grid=(grid,),
      in_specs=[featT_spec] + [feat_spec] * 4 + [s_spec] + [
          full((_D, _H1))] * 5 + [full((4, _H1)), full((_H1, 1)),
          full((_H1, _H2)), full((_H2, 1)), full((_H2, 1)), full((1, 1))],
      out_specs=pl.BlockSpec((1, _TB), lambda i: (0, i)),
      out_shape=jax.ShapeDtypeStruct((1, _B), jnp.float32),
  )(ut, m, a, c, t, s, w1u, w1m, w1a, w1c, w1t, w1s, b1.reshape(_H1, 1),
    W2, b2.reshape(_H2, 1), W3, b3.reshape(1, 1))
  return out


@jax.jit
def kernel(user, movie, actor, country, movie_type, num_reviews,
           normalized_rating, useful_ratings, useless_ratings,
           user_table, movie_table, actor_table, country_table,
           movie_type_table, W1, b1, W2, b2, W3, b3):
  user = user.astype(jnp.int32)
  movie = movie.astype(jnp.int32)
  actor_flat = actor.reshape(-1).astype(jnp.int32)
  country_flat = country.reshape(-1).astype(jnp.int32)
  type_flat = movie_type.reshape(-1).astype(jnp.int32)

  u_rag = jnp.pad(user_table.T[:, _NU - _NU % 128:], ((0, 0), (0, 64)))
  u_lin = _sc_relayout_user(user_table.T, u_rag)
  ut_1d = u_lin.reshape(-1)

  ut, m, a, c, t = _sc_gather_all(
      user, movie, actor_flat, country_flat, type_flat,
      ut_1d, movie_table, actor_table, country_table, movie_type_table)

  s = jnp.stack([num_reviews, normalized_rating, useful_ratings,
                 useless_ratings], axis=1)
  out = _mlp(ut, m, a, c, t, s, W1, b1, W2, b2, W3, b3)
  return out.reshape(-1)


# speedup vs baseline: 8.2831x; 1.0634x over previous
import functools

import jax
import jax.numpy as jnp
from jax import lax
from jax.experimental import pallas as pl
from jax.experimental.pallas import tpu as pltpu
from jax.experimental.pallas import tpu_sc as plsc

_B = 16384
_D = 32
_H1, _H2 = 64, 32
_NU = 1000000

_C = 64
_IDX_BATCH = 128

_TPC = _NU // 128 + 1
_NFULL = 4 * (_TPC - 1)
_UROWS = 4 * _TPC * 8
_GT = 64
_GPC = (_TPC - 1) // _GT
_NG = 4 * _GPC
_NLEFT = _NFULL - _NG * _GT


def _sc_relayout_user(ut_t, u_rag):
  info = plsc.get_sparse_core_info()
  nw = info.num_cores * info.num_subcores
  ngrp = (_NG + nw - 1) // nw

  mesh = plsc.VectorSubcoreMesh(core_axis_name="c", subcore_axis_name="s")

  @functools.partial(
      pl.kernel,
      mesh=mesh,
      out_type=jax.ShapeDtypeStruct((_UROWS, 128), jnp.float32),
      compiler_params=pltpu.CompilerParams(use_tc_tiling_on_sc=True),
      scratch_types=[
          pltpu.VMEM((8, 128 * _GT), jnp.float32),
          pltpu.VMEM((_D, 128), jnp.float32),
          pltpu.SemaphoreType.DMA,
      ],
  )
  def body(ut, ur, uo, buf, tbuf, sem):
    wid = lax.axis_index("s") * info.num_cores + lax.axis_index("c")

    def grp(q, _):
      g = q * nw + wid
      g = jnp.minimum(g, _NG - 1)
      k = g // _GPC
      tc0 = (g % _GPC) * _GT
      pltpu.sync_copy(
          ut.at[pl.ds(pl.multiple_of(k * 8, 8), 8),
                pl.ds(pl.multiple_of(tc0 * 128, 128), 128 * _GT)], buf)
      row0 = (k * _TPC + tc0) * 8
      cps = []
      for t in range(_GT):
        cps.append(pltpu.async_copy(
            buf.at[:, pl.ds(t * 128, 128)],
            uo.at[pl.ds(pl.multiple_of(row0 + t * 8, 8), 8)], sem))
      for cp in cps:
        cp.wait()
      return 0

    lax.fori_loop(0, ngrp, grp, 0)

    nl_pc = _NLEFT // 4
    for w in range(_NLEFT):
      @pl.when(wid == w)
      def _(w=w):
        k = w // nl_pc
        tc = _GPC * _GT + w % nl_pc
        pltpu.sync_copy(
            ut.at[pl.ds(k * 8, 8), pl.ds(tc * 128, 128)],
            buf.at[:, pl.ds(0, 128)])
        pltpu.sync_copy(buf.at[:, pl.ds(0, 128)],
                        uo.at[pl.ds((k * _TPC + tc) * 8, 8)])

    @pl.when(wid == _NLEFT)
    def _():
      pltpu.sync_copy(ur, tbuf)
      for k in range(4):
        pltpu.sync_copy(tbuf.at[pl.ds(k * 8, 8)],
                        uo.at[pl.ds((k * _TPC + _TPC - 1) * 8, 8)])

  return body(ut_t, u_rag)


def _sc_gather_all(user, movie, actor_flat, country_flat, type_flat,
                   ut_1d, movie_tab, actor_tab, country_tab, type_tab):
  info = plsc.get_sparse_core_info()
  nw = info.num_cores * info.num_subcores
  bw = _B // nw
  nchunk = bw // _C

  mesh = plsc.VectorSubcoreMesh(core_axis_name="c", subcore_axis_name="s")
  out_row = jax.ShapeDtypeStruct((_B, _D), jnp.float32)
  out_colT = jax.ShapeDtypeStruct((_D, _B), jnp.float32)

  nbatch = bw * _D // _IDX_BATCH
  nwave = nbatch // 16

  @functools.partial(
      pl.kernel,
      mesh=mesh,
      out_type=[out_colT, out_row, out_row, out_row, out_row],
      compiler_params=pltpu.CompilerParams(use_tc_tiling_on_sc=False),
      scratch_types=[
          pltpu.VMEM((bw,), jnp.int32),
          pltpu.VMEM((16 * _IDX_BATCH,), jnp.int32),
          pltpu.VMEM((bw * _D,), jnp.float32),
          pltpu.VMEM((_C,), jnp.int32),
          pltpu.VMEM((_C * 20,), jnp.int32),
          pltpu.VMEM((_C * 4,), jnp.int32),
          pltpu.VMEM((_C * 4,), jnp.int32),
          pltpu.VMEM((_C, _D), jnp.float32),
          pltpu.VMEM((_C * 20, _D), jnp.float32),
          pltpu.VMEM((_C * 4, _D), jnp.float32),
          pltpu.VMEM((_C * 4, _D), jnp.float32),
          pltpu.VMEM((_C, _D), jnp.float32),
          pltpu.VMEM((_C, _D), jnp.float32),
          pltpu.VMEM((_C, _D), jnp.float32),
          pltpu.SemaphoreType.DMA,
          pltpu.SemaphoreType.DMA,
      ],
  )
  def body(user_i, movie_i, actor_i, country_i, type_i,
           ut, mt, at_, ct, tt,
           uo, mo, ao, co, to,
           idx_u, stage, ubuf,
           idx_m, idx_a, idx_c, idx_t,
           rows_m, rows_a, rows_c, rows_t,
           pool_a, pool_c, pool_t, sem, sem_u):
    wid = lax.axis_index("s") * info.num_cores + lax.axis_index("c")
    base = wid * bw

    pltpu.sync_copy(user_i.at[pl.ds(base, bw)], idx_u)

    def tform(v, _):
      x = idx_u[pl.ds(v * 16, 16)]
      idx_u[pl.ds(v * 16, 16)] = (
          lax.shift_left(lax.shift_right_logical(x, 7), 10)
          + (x & 127))
      return 0
    lax.fori_loop(0, bw // 16, tform, 0)

    def ufire(w):
      cps = []
      for j in range(16):
        k = w * 16 + j
        d = k >> 2
        cst = (d >> 3) * (_TPC * 1024) + (d & 7) * 128
        soff = (k & 3) * _IDX_BATCH
        st = j * _IDX_BATCH
        for v in range(_IDX_BATCH // 16):
          stage[pl.ds(st + v * 16, 16)] = idx_u[pl.ds(soff + v * 16, 16)] + cst
        cps.append(pltpu.async_copy(
            ut.at[stage.at[pl.ds(st, _IDX_BATCH)]],
            ubuf.at[pl.ds(k * _IDX_BATCH, _IDX_BATCH)], sem_u))
      return cps

    def gather(tab, idx_ref, rows_ref, n):
      copies = []
      for off in range(0, n, _IDX_BATCH):
        sz = min(_IDX_BATCH, n - off)
        copies.append(pltpu.async_copy(
            tab.at[idx_ref.at[pl.ds(off, sz)]],
            rows_ref.at[pl.ds(off, sz)], sem))
      return copies

    def pool_loop(rows, pool, n, scale):
      def samp(c, _):
        r = c * n
        a0 = rows[r, pl.ds(0, 16)]
        a1 = rows[r, pl.ds(16, 16)]
        for j in range(1, n):
          a0 = a0 + rows[r + j, pl.ds(0, 16)]
          a1 = a1 + rows[r + j, pl.ds(16, 16)]
        pool[c, pl.ds(0, 16)] = a0 * scale
        pool[c, pl.ds(16, 16)] = a1 * scale
        return 0
      lax.fori_loop(0, _C, samp, 0)

    def chunk(k, _):
      ucps = ufire(k)
      cb = base + k * _C
      pltpu.sync_copy(movie_i.at[pl.ds(cb, _C)], idx_m)
      pltpu.sync_copy(actor_i.at[pl.ds(cb * 20, _C * 20)], idx_a)
      pltpu.sync_copy(country_i.at[pl.ds(cb * 4, _C * 4)], idx_c)
      pltpu.sync_copy(type_i.at[pl.ds(cb * 4, _C * 4)], idx_t)

      cm = gather(mt, idx_m, rows_m, _C)
      ca = gather(at_, idx_a, rows_a, _C * 20)
      cc = gather(ct, idx_c, rows_c, _C * 4)
      ctp = gather(tt, idx_t, rows_t, _C * 4)

      for c in cm:
        c.wait()
      pltpu.sync_copy(rows_m, mo.at[pl.ds(cb, _C)])
      for c in ca:
        c.wait()
      pool_loop(rows_a, pool_a, 20, 1.0 / 20.0)
      for c in cc:
        c.wait()
      pool_loop(rows_c, pool_c, 4, 0.25)
      for c in ctp:
        c.wait()
      pool_loop(rows_t, pool_t, 4, 0.25)

      pltpu.sync_copy(pool_a, ao.at[pl.ds(cb, _C)])
      pltpu.sync_copy(pool_c, co.at[pl.ds(cb, _C)])
      pltpu.sync_copy(pool_t, to.at[pl.ds(cb, _C)])
      for cp in ucps:
        cp.wait()
      return 0

    lax.fori_loop(0, nchunk, chunk, 0)

    def urow(d, _):
      pltpu.sync_copy(ubuf.at[pl.ds(d * bw, bw)], uo.at[d, pl.ds(base, bw)])
      return 0
    lax.fori_loop(0, _D, urow, 0)

  return body(user, movie, actor_flat, country_flat, type_flat,
              ut_1d, movie_tab, actor_tab, country_tab, type_tab)


_TB = 512

_DN0 = (((0,), (0,)), ((), ()))
_DN1 = (((0,), (1,)), ((), ()))


def _mlp_body(ut, m, a, c, t, s, w1u, w1m, w1a, w1c, w1t, w1s, b1,
              w2, b2, w3, b3, o):
  h = (lax.dot_general(w1u[...], ut[...], _DN0)
       + lax.dot_general(w1m[...], m[...], _DN1)
       + lax.dot_general(w1a[...], a[...], _DN1)
       + lax.dot_general(w1c[...], c[...], _DN1)
       + lax.dot_general(w1t[...], t[...], _DN1)
       + lax.dot_general(w1s[...], s[...], _DN1)
       + b1[...])
  h = jnp.maximum(h, 0.0)
  h = jnp.maximum(lax.dot_general(w2[...], h, _DN0) + b2[...], 0.0)
  o[...] = lax.dot_general(w3[...], h, _DN0) + b3[...]


def _mlp(ut, m, a, c, t, s, W1, b1, W2, b2, W3, b3):
  grid = _B // _TB
  featT_spec = pl.BlockSpec((_D, _TB), lambda i: (0, i))
  feat_spec = pl.BlockSpec((_TB, _D), lambda i: (i, 0))
  s_spec = pl.BlockSpec((_TB, 4), lambda i: (i, 0))

  def full(shape):
    return pl.BlockSpec(shape, lambda i: tuple(0 for _ in shape))

  w1u, w1m, w1a, w1c, w1t = (W1[k * _D:(k + 1) * _D] for k in range(5))
  w1s = W1[5 * _D:]
  out = pl.pallas_call(
      _mlp_body,
      grid=(grid,),
      in_specs=[featT_spec] + [feat_spec] * 4 + [s_spec] + [
          full((_D, _H1))] * 5 + [full((4, _H1)), full((_H1, 1)),
          full((_H1, _H2)), full((_H2, 1)), full((_H2, 1)), full((1, 1))],
      out_specs=pl.BlockSpec((1, _TB), lambda i: (0, i)),
      out_shape=jax.ShapeDtypeStruct((1, _B), jnp.float32),
  )(ut, m, a, c, t, s, w1u, w1m, w1a, w1c, w1t, w1s, b1.reshape(_H1, 1),
    W2, b2.reshape(_H2, 1), W3, b3.reshape(1, 1))
  return out


@jax.jit
def kernel(user, movie, actor, country, movie_type, num_reviews,
           normalized_rating, useful_ratings, useless_ratings,
           user_table, movie_table, actor_table, country_table,
           movie_type_table, W1, b1, W2, b2, W3, b3):
  user = user.astype(jnp.int32)
  movie = movie.astype(jnp.int32)
  actor_flat = actor.reshape(-1).astype(jnp.int32)
  country_flat = country.reshape(-1).astype(jnp.int32)
  type_flat = movie_type.reshape(-1).astype(jnp.int32)

  u_rag = jnp.pad(user_table.T[:, _NU - _NU % 128:], ((0, 0), (0, 64)))
  u_lin = _sc_relayout_user(user_table.T, u_rag)
  ut_1d = u_lin.reshape(-1)

  ut, m, a, c, t = _sc_gather_all(
      user, movie, actor_flat, country_flat, type_flat,
      ut_1d, movie_table, actor_table, country_table, movie_type_table)

  s = jnp.stack([num_reviews, normalized_rating, useful_ratings,
                 useless_ratings], axis=1)
  out = _mlp(ut, m, a, c, t, s, W1, b1, W2, b2, W3, b3)
  return out.reshape(-1)
